# Initial kernel scaffold; baseline (speedup 1.0000x reference)
import jax
import jax.numpy as jnp
from jax.experimental import pallas as pl


def kernel(uv, latent_hi, latent_lo):
    raise NotImplementedError("write your pallas kernel here")


# speedup vs baseline: 17.6380x; 17.6380x over previous
import functools

import jax
import jax.numpy as jnp
from jax import lax
from jax.experimental import pallas as pl
from jax.experimental.pallas import tpu as pltpu
from jax.experimental.pallas import tpu_sc as plsc

_B = 1048576
_NW = 32
_PERW = _B // _NW
_NQ = 512
_NBLK = _PERW // _NQ
_NG = _NQ // 16
_IDXR = _NQ // 128


def _quant_body(x_ref, o_ref, *, qmax):
    x = x_ref[...]
    o_ref[...] = jnp.round(jnp.clip(x, 0.0, 1.0) * qmax) / qmax


def _quantize(x, qmax, hb):
    c, h, w = x.shape
    return pl.pallas_call(
        functools.partial(_quant_body, qmax=qmax),
        grid=(c, h // hb),
        in_specs=[pl.BlockSpec((1, hb, w), lambda i, j: (i, j, 0))],
        out_specs=pl.BlockSpec((1, hb, w), lambda i, j: (i, j, 0)),
        out_shape=jax.ShapeDtypeStruct((c, h, w), jnp.float32),
    )(x)


def _sc_sample(u, v, hi_t, lo_t):
    mesh = plsc.VectorSubcoreMesh(core_axis_name="c", subcore_axis_name="s")

    @functools.partial(
        pl.kernel,
        mesh=mesh,
        out_type=jax.ShapeDtypeStruct((_B, 24), jnp.float32),
        scratch_types=(
            [pltpu.VMEM((_NQ,), jnp.float32)] * 2
            + [pltpu.VMEM((_IDXR, 128), jnp.int32)] * 8
            + [pltpu.VMEM((_NQ,), jnp.float32)] * 8
            + [pltpu.VMEM((_NQ, 8), jnp.float32)] * 4
            + [pltpu.VMEM((_NQ, 16), jnp.float32)] * 4
            + [pltpu.VMEM((_NQ, 24), jnp.float32),
               pltpu.SemaphoreType.DMA]
        ),
        compiler_params=pltpu.CompilerParams(
            needs_layout_passes=False, use_tc_tiling_on_sc=False),
    )
    def run(u_hbm, v_hbm, hi_hbm, lo_hbm, out_hbm,
            u_v, v_v,
            ih00, ih01, ih10, ih11, il00, il01, il10, il11,
            wh00, wh01, wh10, wh11, wl00, wl01, wl10, wl11,
            rh00, rh01, rh10, rh11, rl00, rl01, rl10, rl11,
            out_v, sem):
        wid = lax.axis_index("s") * 2 + lax.axis_index("c")
        base = wid * _PERW

        def block(b, carry):
            qb = base + b * _NQ
            pltpu.sync_copy(u_hbm.at[pl.ds(qb, _NQ)], u_v)
            pltpu.sync_copy(v_hbm.at[pl.ds(qb, _NQ)], v_v)

            def prep(i, c2):
                s = pl.ds(i * 16, 16)
                r = i // 8
                cs = pl.ds((i % 8) * 16, 16)
                uu = u_v[s]
                vv = v_v[s]
                xh = uu * 2047.0
                yh = vv * 2047.0
                xi = jnp.minimum(xh.astype(jnp.int32), 2046)
                yi = jnp.minimum(yh.astype(jnp.int32), 2046)
                fx = xh - xi.astype(jnp.float32)
                fy = yh - yi.astype(jnp.float32)
                t = yi * 2048 + xi
                ih00[r, cs] = t
                ih01[r, cs] = t + 1
                ih10[r, cs] = t + 2048
                ih11[r, cs] = t + 2049
                gx = 1.0 - fx
                gy = 1.0 - fy
                wh00[s] = gy * gx
                wh01[s] = gy * fx
                wh10[s] = fy * gx
                wh11[s] = fy * fx
                xl = uu * 255.0
                yl = vv * 255.0
                xli = jnp.minimum(xl.astype(jnp.int32), 254)
                yli = jnp.minimum(yl.astype(jnp.int32), 254)
                flx = xl - xli.astype(jnp.float32)
                fly = yl - yli.astype(jnp.float32)
                tl = yli * 256 + xli
                il00[r, cs] = tl
                il01[r, cs] = tl + 1
                il10[r, cs] = tl + 256
                il11[r, cs] = tl + 257
                glx = 1.0 - flx
                gly = 1.0 - fly
                wl00[s] = gly * glx
                wl01[s] = gly * flx
                wl10[s] = fly * glx
                wl11[s] = fly * flx
                return c2

            lax.fori_loop(0, _NG, prep, 0)

            cps = []
            for j in range(_IDXR):
                dh = pl.ds(j * 128, 128)
                dl = pl.ds(j * 128, 128)
                cps.append(pltpu.async_copy(hi_hbm.at[ih00.at[j]], rh00.at[dh], sem))
                cps.append(pltpu.async_copy(hi_hbm.at[ih01.at[j]], rh01.at[dh], sem))
                cps.append(pltpu.async_copy(hi_hbm.at[ih10.at[j]], rh10.at[dh], sem))
                cps.append(pltpu.async_copy(hi_hbm.at[ih11.at[j]], rh11.at[dh], sem))
                cps.append(pltpu.async_copy(lo_hbm.at[il00.at[j]], rl00.at[dl], sem))
                cps.append(pltpu.async_copy(lo_hbm.at[il01.at[j]], rl01.at[dl], sem))
                cps.append(pltpu.async_copy(lo_hbm.at[il10.at[j]], rl10.at[dl], sem))
                cps.append(pltpu.async_copy(lo_hbm.at[il11.at[j]], rl11.at[dl], sem))
            for cp in cps:
                cp.wait()

            def comb(i, c2):
                s = pl.ds(i * 16, 16)
                qv = i * 16 + lax.iota(jnp.int32, 16)
                w00 = wh00[s]
                w01 = wh01[s]
                w10 = wh10[s]
                w11 = wh11[s]
                for c in range(8):
                    cv = jnp.full((16,), c, jnp.int32)
                    val = (plsc.load_gather(rh00, [qv, cv]) * w00
                           + plsc.load_gather(rh01, [qv, cv]) * w01
                           + plsc.load_gather(rh10, [qv, cv]) * w10
                           + plsc.load_gather(rh11, [qv, cv]) * w11)
                    plsc.store_scatter(out_v, [qv, cv], val)
                m00 = wl00[s]
                m01 = wl01[s]
                m10 = wl10[s]
                m11 = wl11[s]
                for c in range(16):
                    cv = jnp.full((16,), c, jnp.int32)
                    ov = jnp.full((16,), 8 + c, jnp.int32)
                    val = (plsc.load_gather(rl00, [qv, cv]) * m00
                           + plsc.load_gather(rl01, [qv, cv]) * m01
                           + plsc.load_gather(rl10, [qv, cv]) * m10
                           + plsc.load_gather(rl11, [qv, cv]) * m11)
                    plsc.store_scatter(out_v, [qv, ov], val)
                return c2

            lax.fori_loop(0, _NG, comb, 0)
            pltpu.sync_copy(out_v, out_hbm.at[pl.ds(qb, _NQ)])
            return carry

        lax.fori_loop(0, _NBLK, block, 0)

    return run(u, v, hi_t, lo_t)


def kernel(uv, latent_hi, latent_lo):
    hi = latent_hi[0]
    lo = latent_lo[0]
    hi_q = _quantize(hi, 255.0, 512)
    lo_q = _quantize(lo, 15.0, 256)
    hi_t = hi_q.transpose(1, 2, 0).reshape(2048 * 2048, 8)
    lo_t = lo_q.transpose(1, 2, 0).reshape(256 * 256, 16)
    u = uv[:, 0]
    v = uv[:, 1]
    return _sc_sample(u, v, hi_t, lo_t)


# speedup vs baseline: 30.2165x; 1.7131x over previous
import functools

import jax
import jax.numpy as jnp
from jax import lax
from jax.experimental import pallas as pl
from jax.experimental.pallas import tpu as pltpu
from jax.experimental.pallas import tpu_sc as plsc

_B = 1048576
_NW = 32
_PERW = _B // _NW
_NQ = 256
_NBLK = _PERW // _NQ
_NG = _NQ // 16
_IDXR = _NQ // 128


def _quant_body(x_ref, o_ref, *, qmax):
    x = x_ref[...]
    o_ref[...] = jnp.round(jnp.clip(x, 0.0, 1.0) * qmax) / qmax


def _quantize(x, qmax, hb):
    c, h, w = x.shape
    return pl.pallas_call(
        functools.partial(_quant_body, qmax=qmax),
        grid=(c, h // hb),
        in_specs=[pl.BlockSpec((1, hb, w), lambda i, j: (i, j, 0))],
        out_specs=pl.BlockSpec((1, hb, w), lambda i, j: (i, j, 0)),
        out_shape=jax.ShapeDtypeStruct((c, h, w), jnp.float32),
    )(x)


def _sc_transpose(hi_q, lo_q):
    mesh = plsc.VectorSubcoreMesh(core_axis_name="c", subcore_axis_name="s")

    @functools.partial(
        pl.kernel,
        mesh=mesh,
        out_type=(jax.ShapeDtypeStruct((2048 * 2048 * 8,), jnp.float32),
                  jax.ShapeDtypeStruct((256 * 256 * 16,), jnp.float32)),
        scratch_types=(
            [pltpu.VMEM((2048,), jnp.float32)] * 8
            + [pltpu.VMEM((16384,), jnp.float32)]
            + [pltpu.VMEM((256,), jnp.float32)] * 16
            + [pltpu.VMEM((4096,), jnp.float32)]
            + [pltpu.SemaphoreType.DMA]
        ),
        compiler_params=pltpu.CompilerParams(
            needs_layout_passes=False, use_tc_tiling_on_sc=False),
    )
    def run(hi_hbm, lo_hbm, hit_hbm, lot_hbm,
            c0, c1, c2, c3, c4, c5, c6, c7, ob,
            d0, d1, d2, d3, d4, d5, d6, d7, d8, d9, d10, d11, d12, d13,
            d14, d15, lb, sem):
        wid = lax.axis_index("s") * 2 + lax.axis_index("c")
        hcb = [c0, c1, c2, c3, c4, c5, c6, c7]
        lcb = [d0, d1, d2, d3, d4, d5, d6, d7, d8, d9, d10, d11, d12, d13,
               d14, d15]

        def hrow(r, carry):
            y = wid * 64 + r
            cps = [pltpu.async_copy(hi_hbm.at[c, y], hcb[c], sem)
                   for c in range(8)]
            for cp in cps:
                cp.wait()

            def grp(g, c2_):
                t16 = g * 16 + lax.iota(jnp.int32, 16)
                s = pl.ds(g * 16, 16)
                for c in range(8):
                    plsc.store_scatter(ob, [t16 * 8 + c], hcb[c][s])
                return c2_

            lax.fori_loop(0, 128, grp, 0)
            pltpu.sync_copy(ob, hit_hbm.at[pl.ds(y * 16384, 16384)])
            return carry

        lax.fori_loop(0, 64, hrow, 0)

        def lrow(r, carry):
            y = wid * 8 + r
            cps = [pltpu.async_copy(lo_hbm.at[c, y], lcb[c], sem)
                   for c in range(16)]
            for cp in cps:
                cp.wait()

            def grp(g, c2_):
                t16 = g * 16 + lax.iota(jnp.int32, 16)
                s = pl.ds(g * 16, 16)
                for c in range(16):
                    plsc.store_scatter(lb, [t16 * 16 + c], lcb[c][s])
                return c2_

            lax.fori_loop(0, 16, grp, 0)
            pltpu.sync_copy(lb, lot_hbm.at[pl.ds(y * 4096, 4096)])
            return carry

        lax.fori_loop(0, 8, lrow, 0)

    return run(hi_q, lo_q)


def _slot_types():
    return ([pltpu.VMEM((_NQ,), jnp.float32)] * 2
            + [pltpu.VMEM((_IDXR, 128), jnp.int32)] * 8
            + [pltpu.VMEM((_NQ,), jnp.float32)] * 8
            + [pltpu.VMEM((_NQ, 8), jnp.float32)] * 4
            + [pltpu.VMEM((_NQ, 16), jnp.float32)] * 4
            + [pltpu.VMEM((_NQ, 24), jnp.float32)])


def _sc_sample(u, v, hi_t, lo_t):
    mesh = plsc.VectorSubcoreMesh(core_axis_name="c", subcore_axis_name="s")

    @functools.partial(
        pl.kernel,
        mesh=mesh,
        out_type=jax.ShapeDtypeStruct((_B, 24), jnp.float32),
        scratch_types=([_slot_types(), _slot_types()]
                       + [pltpu.SemaphoreType.DMA] * 6),
        compiler_params=pltpu.CompilerParams(
            needs_layout_passes=False, use_tc_tiling_on_sc=False),
    )
    def run(u_hbm, v_hbm, hi_hbm, lo_hbm, out_hbm,
            slot0, slot1, sg0, sg1, su0, su1, so0, so1):
        slots = [slot0, slot1]
        sgs = [sg0, sg1]
        sus = [su0, su1]
        sos = [so0, so1]
        wid = lax.axis_index("s") * 2 + lax.axis_index("c")
        base = wid * _PERW

        def uv_cps(si, b):
            qb = base + b * _NQ
            return [pltpu.make_async_copy(u_hbm.at[pl.ds(qb, _NQ)],
                                          slots[si][0], sus[si]),
                    pltpu.make_async_copy(v_hbm.at[pl.ds(qb, _NQ)],
                                          slots[si][1], sus[si])]

        def gather_cps(si):
            sl = slots[si]
            idx = sl[2:10]
            rh = sl[18:22]
            rl = sl[22:26]
            cps = []
            for j in range(_IDXR):
                d = pl.ds(j * 128, 128)
                for a in range(4):
                    cps.append(pltpu.make_async_copy(
                        hi_hbm.at[idx[a].at[j]], rh[a].at[d], sgs[si]))
                for a in range(4):
                    cps.append(pltpu.make_async_copy(
                        lo_hbm.at[idx[4 + a].at[j]], rl[a].at[d], sgs[si]))
            return cps

        def out_cp(si, b):
            qb = base + b * _NQ
            return pltpu.make_async_copy(slots[si][26],
                                         out_hbm.at[pl.ds(qb, _NQ)], sos[si])

        def prep(si):
            sl = slots[si]
            u_v, v_v = sl[0], sl[1]
            ih00, ih01, ih10, ih11, il00, il01, il10, il11 = sl[2:10]
            wh00, wh01, wh10, wh11, wl00, wl01, wl10, wl11 = sl[10:18]

            def body(i, c2):
                s = pl.ds(i * 16, 16)
                r = i // 8
                cs = pl.ds((i % 8) * 16, 16)
                uu = u_v[s]
                vv = v_v[s]
                xh = uu * 2047.0
                yh = vv * 2047.0
                xi = jnp.minimum(xh.astype(jnp.int32), 2046)
                yi = jnp.minimum(yh.astype(jnp.int32), 2046)
                fx = xh - xi.astype(jnp.float32)
                fy = yh - yi.astype(jnp.float32)
                t = yi * 2048 + xi
                ih00[r, cs] = t
                ih01[r, cs] = t + 1
                ih10[r, cs] = t + 2048
                ih11[r, cs] = t + 2049
                gx = 1.0 - fx
                gy = 1.0 - fy
                wh00[s] = gy * gx
                wh01[s] = gy * fx
                wh10[s] = fy * gx
                wh11[s] = fy * fx
                xl = uu * 255.0
                yl = vv * 255.0
                xli = jnp.minimum(xl.astype(jnp.int32), 254)
                yli = jnp.minimum(yl.astype(jnp.int32), 254)
                flx = xl - xli.astype(jnp.float32)
                fly = yl - yli.astype(jnp.float32)
                tl = yli * 256 + xli
                il00[r, cs] = tl
                il01[r, cs] = tl + 1
                il10[r, cs] = tl + 256
                il11[r, cs] = tl + 257
                glx = 1.0 - flx
                gly = 1.0 - fly
                wl00[s] = gly * glx
                wl01[s] = gly * flx
                wl10[s] = fly * glx
                wl11[s] = fly * flx
                return c2

            lax.fori_loop(0, _NG, body, 0)

        def comb(si):
            sl = slots[si]
            wh00, wh01, wh10, wh11, wl00, wl01, wl10, wl11 = sl[10:18]
            rh00, rh01, rh10, rh11 = sl[18:22]
            rl00, rl01, rl10, rl11 = sl[22:26]
            out_v = sl[26]

            def body(i, c2):
                s = pl.ds(i * 16, 16)
                qv = i * 16 + lax.iota(jnp.int32, 16)
                w00 = wh00[s]
                w01 = wh01[s]
                w10 = wh10[s]
                w11 = wh11[s]
                for c in range(8):
                    cv = jnp.full((16,), c, jnp.int32)
                    val = (plsc.load_gather(rh00, [qv, cv]) * w00
                           + plsc.load_gather(rh01, [qv, cv]) * w01
                           + plsc.load_gather(rh10, [qv, cv]) * w10
                           + plsc.load_gather(rh11, [qv, cv]) * w11)
                    plsc.store_scatter(out_v, [qv, cv], val)
                m00 = wl00[s]
                m01 = wl01[s]
                m10 = wl10[s]
                m11 = wl11[s]
                for c in range(16):
                    cv = jnp.full((16,), c, jnp.int32)
                    ov = jnp.full((16,), 8 + c, jnp.int32)
                    val = (plsc.load_gather(rl00, [qv, cv]) * m00
                           + plsc.load_gather(rl01, [qv, cv]) * m01
                           + plsc.load_gather(rl10, [qv, cv]) * m10
                           + plsc.load_gather(rl11, [qv, cv]) * m11)
                    plsc.store_scatter(out_v, [qv, ov], val)
                return c2

            lax.fori_loop(0, _NG, body, 0)

        for cp in uv_cps(0, 0):
            cp.start()
        for cp in uv_cps(0, 0):
            cp.wait()
        prep(0)
        for cp in gather_cps(0):
            cp.start()
        for cp in uv_cps(1, 1):
            cp.start()

        def pair(k, carry):
            b0 = 2 * k
            b1 = b0 + 1
            b2 = b0 + 2
            b3 = b0 + 3

            for cp in uv_cps(1, b1):
                cp.wait()
            prep(1)
            for cp in gather_cps(1):
                cp.start()

            @pl.when(b2 < _NBLK)
            def _():
                for cp in uv_cps(0, b2):
                    cp.start()

            for cp in gather_cps(0):
                cp.wait()

            @pl.when(k > 0)
            def _():
                out_cp(0, 0).wait()

            comb(0)
            out_cp(0, b0).start()

            @pl.when(b2 < _NBLK)
            def _():
                for cp in uv_cps(0, b2):
                    cp.wait()
                prep(0)
                for cp in gather_cps(0):
                    cp.start()

            @pl.when(b3 < _NBLK)
            def _():
                for cp in uv_cps(1, b3):
                    cp.start()

            for cp in gather_cps(1):
                cp.wait()

            @pl.when(k > 0)
            def _():
                out_cp(1, 0).wait()

            comb(1)
            out_cp(1, b1).start()
            return carry

        lax.fori_loop(0, _NBLK // 2, pair, 0)
        out_cp(0, 0).wait()
        out_cp(1, 0).wait()

    return run(u, v, hi_t, lo_t)


def kernel(uv, latent_hi, latent_lo):
    hi = latent_hi[0]
    lo = latent_lo[0]
    hi_q = _quantize(hi, 255.0, 512)
    lo_q = _quantize(lo, 15.0, 256)
    hi_f, lo_f = _sc_transpose(hi_q, lo_q)
    hi_t = hi_f.reshape(2048 * 2048, 8)
    lo_t = lo_f.reshape(256 * 256, 16)
    u = uv[:, 0]
    v = uv[:, 1]
    return _sc_sample(u, v, hi_t, lo_t)
